# baseline (speedup 1.0000x reference)
import functools

import jax
import jax.numpy as jnp
from jax import lax
from jax.experimental import pallas as pl
from jax.experimental.pallas import tpu as pltpu
from jax.experimental.pallas import tpu_sc as plsc

_B = 1024
_P = 20
_V = 20
_D = 18432
_N = _B * _P
_NCOL = 4
_NROW = 8
_DS = _D // _NCOL
_RPW = _N // _NROW
_NGRP = _RPW // 16

_mesh = plsc.VectorSubcoreMesh(core_axis_name="c", subcore_axis_name="s")


@functools.partial(
    pl.kernel,
    out_type=jax.ShapeDtypeStruct((_N, _D), jnp.float32),
    mesh=_mesh,
    scratch_types=[
        pltpu.VMEM((_RPW,), jnp.int32),
        pltpu.VMEM((_V, _DS), jnp.float32),
        pltpu.SemaphoreType.DMA,
    ],
    compiler_params=pltpu.CompilerParams(use_tc_tiling_on_sc=False),
)
def _sc_lookup(idx_hbm, table_hbm, out_hbm, idx_v, tslice, psem):
    cid = lax.axis_index("c")
    sid = lax.axis_index("s")
    colg = sid // _NCOL
    rowg = (sid % _NCOL) + _NCOL * cid
    col0 = colg * _DS
    row0 = rowg * _RPW

    pltpu.sync_copy(table_hbm.at[:, pl.ds(col0, _DS)], tslice)
    pltpu.sync_copy(idx_hbm.at[rowg], idx_v)

    def grp_body(g, carry):
        vec = idx_v[pl.ds(g * 16, 16)]
        r0 = row0 + g * 16
        for l in range(16):
            pltpu.make_async_copy(
                tslice.at[pl.ds(vec[l], 1)],
                out_hbm.at[pl.ds(r0 + l, 1), pl.ds(col0, _DS)],
                psem).start()
        return carry

    lax.fori_loop(0, _NGRP, grp_body, 0)

    def drain(j, carry):
        pltpu.make_async_copy(
            tslice.at[pl.ds(0, 1)],
            out_hbm.at[pl.ds(row0, 1), pl.ds(col0, _DS)],
            psem).wait()
        return carry

    lax.fori_loop(0, _RPW, drain, 0)


def kernel(prefix, embedding_table):
    idx2 = prefix.reshape(_NROW, _RPW)
    out = _sc_lookup(idx2, embedding_table)
    return out.reshape(_B, _P, _D)


# speedup vs baseline: 1.0083x; 1.0083x over previous
import functools

import jax
import jax.numpy as jnp
from jax import lax
from jax.experimental import pallas as pl
from jax.experimental.pallas import tpu as pltpu
from jax.experimental.pallas import tpu_sc as plsc

_B = 1024
_P = 20
_V = 20
_D = 18432
_N = _B * _P
_NCOL = 4
_NROW = 8
_DS = _D // _NCOL
_RPW = _N // _NROW
_NGRP = _RPW // 16

_mesh = plsc.VectorSubcoreMesh(core_axis_name="c", subcore_axis_name="s")


@functools.partial(
    pl.kernel,
    out_type=jax.ShapeDtypeStruct((_N, _D), jnp.float32),
    mesh=_mesh,
    scratch_types=[
        pltpu.VMEM((_RPW,), jnp.int32),
        pltpu.VMEM_SHARED((_V, _DS), jnp.float32),
        pltpu.SemaphoreType.DMA,
    ],
)
def _sc_lookup(idx_hbm, table_hbm, out_hbm, idx_v, tslice, psem):
    cid = lax.axis_index("c")
    sid = lax.axis_index("s")
    colg = sid // _NCOL
    rowg = (sid % _NCOL) + _NCOL * cid
    col0 = colg * _DS
    row0 = rowg * _RPW

    pltpu.sync_copy(table_hbm.at[:, pl.ds(col0, _DS)], tslice)
    pltpu.sync_copy(idx_hbm.at[rowg], idx_v)

    def grp_body(g, carry):
        vec = idx_v[pl.ds(g * 16, 16)]
        r0 = row0 + g * 16
        for l in range(16):
            pltpu.make_async_copy(
                tslice.at[pl.ds(vec[l], 1)],
                out_hbm.at[pl.ds(r0 + l, 1), pl.ds(col0, _DS)],
                psem).start()
        return carry

    lax.fori_loop(0, _NGRP, grp_body, 0)

    def drain(j, carry):
        pltpu.make_async_copy(
            tslice.at[pl.ds(0, 1)],
            out_hbm.at[pl.ds(row0, 1), pl.ds(col0, _DS)],
            psem).wait()
        return carry

    lax.fori_loop(0, _RPW, drain, 0)


def kernel(prefix, embedding_table):
    idx2 = prefix.reshape(_NROW, _RPW)
    out = _sc_lookup(idx2, embedding_table)
    return out.reshape(_B, _P, _D)


# speedup vs baseline: 1.0984x; 1.0894x over previous
import functools

import jax
import jax.numpy as jnp
from jax import lax
from jax.experimental import pallas as pl
from jax.experimental.pallas import tpu as pltpu
from jax.experimental.pallas import tpu_sc as plsc

_B = 1024
_P = 20
_V = 20
_D = 18432
_N = _B * _P
_NCOL = 4
_NROW = 8
_DS = _D // _NCOL
_RPW = _N // _NROW
_NGRP = _RPW // 16

_mesh = plsc.VectorSubcoreMesh(core_axis_name="c", subcore_axis_name="s")


@functools.partial(
    pl.kernel,
    out_type=jax.ShapeDtypeStruct((_N, _D), jnp.float32),
    mesh=_mesh,
    scratch_types=[
        pltpu.VMEM((_RPW,), jnp.int32),
        pltpu.VMEM((_V, _DS), jnp.float32),
        pltpu.SemaphoreType.DMA,
    ],
)
def _sc_lookup(idx_hbm, table_hbm, out_hbm, idx_v, tslice, psem):
    cid = lax.axis_index("c")
    sid = lax.axis_index("s")
    colg = sid // _NCOL
    rowg = (sid % _NCOL) + _NCOL * cid
    col0 = colg * _DS
    row0 = rowg * _RPW

    pltpu.sync_copy(table_hbm.at[:, pl.ds(col0, _DS)], tslice)
    pltpu.sync_copy(idx_hbm.at[rowg], idx_v)

    def grp_body(g, carry):
        vec = idx_v[pl.ds(g * 16, 16)]
        r0 = row0 + g * 16
        for l in range(16):
            pltpu.make_async_copy(
                tslice.at[pl.ds(vec[l], 1)],
                out_hbm.at[pl.ds(r0 + l, 1), pl.ds(col0, _DS)],
                psem).start()
        return carry

    lax.fori_loop(0, _NGRP, grp_body, 0)

    def drain(j, carry):
        pltpu.make_async_copy(
            tslice.at[pl.ds(0, 1)],
            out_hbm.at[pl.ds(row0, 1), pl.ds(col0, _DS)],
            psem).wait()
        return carry

    lax.fori_loop(0, _RPW, drain, 0)


def kernel(prefix, embedding_table):
    idx2 = prefix.reshape(_NROW, _RPW)
    out = _sc_lookup(idx2, embedding_table)
    return out.reshape(_B, _P, _D)
